# Initial kernel scaffold; baseline (speedup 1.0000x reference)
import jax
import jax.numpy as jnp
from jax.experimental import pallas as pl


def kernel(x, edge_index, edge_type, rel_embed, loop_rel, W_in, W_out, W_loop, W_rel, bias):
    raise NotImplementedError("write your pallas kernel here")


# speedup vs baseline: 1.4186x; 1.4186x over previous
import functools

import jax
import jax.numpy as jnp
from jax import lax
from jax.experimental import pallas as pl
from jax.experimental.pallas import tpu as pltpu
from jax.experimental.pallas import tpu_sc as plsc

N = 10000
NPAD = 10240
NR = 5120
NACC = 5248
D = 256
DH = 128
NCB = 2
EH = 80000
EPT = 5120
NCH = 40
C = 128
RROWS = 400


def _rsqrt_newton(v):
    i = lax.bitcast_convert_type(v, jnp.int32)
    i = jnp.int32(0x5F3759DF) - (i >> 1)
    y = lax.bitcast_convert_type(i, jnp.float32)
    for _ in range(3):
        y = y * (1.5 - 0.5 * v * y * y)
    return y


def _sc_body(src_hbm, dst_hbm, et_hbm, x0_hbm, x1_hbm, nrel0_hbm, nrel1_hbm,
             agg_hbm,
             src_i, dst_i, et_i, dst2_i, norm_i, nsv, ndv, xrows,
             ones_v, zdeg, deg_sp, acc_sp, sem):
    s = lax.axis_index("s")

    zeros16 = jnp.zeros((16,), jnp.float32)

    def _ov(i, _):
        ones_v[pl.ds(i * 16, 16)] = zeros16 + 1.0
        return 0
    lax.fori_loop(0, C // 16, _ov, 0)

    for h in range(2):
        pltpu.sync_copy(src_hbm.at[h, s], src_i)
        pltpu.sync_copy(dst_hbm.at[h, s], dst_i)
        pltpu.sync_copy(et_hbm.at[h, s], et_i)

        def _zv(i, _):
            zdeg[pl.ds(i * 16, 16)] = zeros16
            return 0
        lax.fori_loop(0, 640 // 16, _zv, 0)

        pltpu.sync_copy(zdeg, deg_sp.at[pl.ds(s * 640, 640)])
        plsc.subcore_barrier()

        def _dega(ch, _):
            pltpu.sync_copy(ones_v, deg_sp.at[dst_i.at[ch]], add=True)
            return 0
        lax.fori_loop(0, NCH, _dega, 0)
        plsc.subcore_barrier()

        pltpu.sync_copy(deg_sp.at[pl.ds(s * 640, 640)], zdeg)
        plsc.subcore_barrier()

        def _newt(i, _):
            v = zdeg[pl.ds(i * 16, 16)]
            live = v >= 0.5
            y = _rsqrt_newton(jnp.maximum(v, 1.0))
            zdeg[pl.ds(i * 16, 16)] = jnp.where(live, y, 0.0)
            return 0
        lax.fori_loop(0, 640 // 16, _newt, 0)
        pltpu.sync_copy(zdeg, deg_sp.at[pl.ds(s * 640, 640)])
        plsc.subcore_barrier()

        def _norm(ch, _):
            pltpu.async_copy(deg_sp.at[src_i.at[ch]], nsv, sem).wait()
            pltpu.async_copy(deg_sp.at[dst_i.at[ch]], ndv, sem).wait()
            for j in range(8):
                norm_i[ch, pl.ds(j * 16, 16)] = (
                    nsv[pl.ds(j * 16, 16)] * ndv[pl.ds(j * 16, 16)])
            return 0
        lax.fori_loop(0, NCH, _norm, 0)

        for cb in range(NCB):
            xh = (x0_hbm, x1_hbm)[cb]
            nrelh = (nrel0_hbm, nrel1_hbm)[cb]

            def _pass(p, _0):
                lo = p * NR

                def _remap(i, _):
                    ch = i // 8
                    j = i % 8
                    dv = dst_i[ch, pl.ds(j * 16, 16)] - lo
                    ok = (dv >= 0) & (dv < NR)
                    dst2_i[ch, pl.ds(j * 16, 16)] = jnp.where(ok, dv, NR)
                    return 0
                lax.fori_loop(0, NCH * 8, _remap, 0)

                def _zx(i, _):
                    xrows[i // 8, pl.ds((i % 8) * 16, 16)] = zeros16
                    return 0
                lax.fori_loop(0, C * 8, _zx, 0)
                pltpu.sync_copy(xrows, acc_sp.at[pl.ds(s * C, C)])
                pltpu.sync_copy(xrows, acc_sp.at[pl.ds((s + 16) * C, C)])

                @pl.when(s < 9)
                def _():
                    pltpu.sync_copy(xrows, acc_sp.at[pl.ds((s + 32) * C, C)])
                plsc.subcore_barrier()

                def _chunk(ch, _):
                    pltpu.async_copy(xh.at[src_i.at[ch]], xrows, sem).wait()
                    pltpu.async_copy(nrelh.at[et_i.at[ch]], xrows, sem,
                                     add=True).wait()

                    def _grp(g, _2):
                        nv = norm_i[ch, pl.ds(g * 16, 16)]
                        for eo in range(16):
                            nrm = nv[eo]
                            row = g * 16 + eo
                            for j in range(8):
                                xv = xrows[row, pl.ds(j * 16, 16)]
                                xrows[row, pl.ds(j * 16, 16)] = xv * nrm
                        return 0
                    lax.fori_loop(0, 8, _grp, 0)

                    pltpu.sync_copy(xrows, acc_sp.at[dst2_i.at[ch]], add=True)
                    return 0
                lax.fori_loop(0, NCH, _chunk, 0)
                plsc.subcore_barrier()

                pltpu.sync_copy(
                    acc_sp.at[pl.ds(s * 320, 320)],
                    agg_hbm.at[h, cb, pl.ds(lo + s * 320, 320)])
                plsc.subcore_barrier()
                return 0
            lax.fori_loop(0, 2, _pass, 0)


def _tc_node_body(ain_ref, aout_ref, x_ref, win_ref, wout_ref, wloop_ref,
                  lr_ref, b_ref, out_ref):
    acc = jnp.dot(ain_ref[...], win_ref[...],
                  preferred_element_type=jnp.float32)
    acc += jnp.dot(aout_ref[...], wout_ref[...],
                   preferred_element_type=jnp.float32)
    acc += jnp.dot(x_ref[...] - lr_ref[...], wloop_ref[...],
                   preferred_element_type=jnp.float32)
    out_ref[...] = jnp.tanh(acc * (1.0 / 3.0) + b_ref[...])


def _tc_rel_body(rel_ref, w_ref, out_ref):
    out_ref[...] = jnp.dot(rel_ref[...], w_ref[...],
                           preferred_element_type=jnp.float32)


def kernel(x, edge_index, edge_type, rel_embed, loop_rel, W_in, W_out,
           W_loop, W_rel, bias):
    half = EH

    def prep(idx, pad):
        a = idx.reshape(2, 16, EH // 16)
        p = jnp.full((2, 16, EPT - EH // 16), pad, jnp.int32)
        return jnp.concatenate([a, p], axis=-1).reshape(2, 16, NCH, C)

    src_a = prep(edge_index[0].reshape(2, half), 0)
    dst_a = prep(edge_index[1].reshape(2, half), N)
    et_a = prep(edge_type.reshape(2, half), 0)

    xs = [x[:, i * DH:(i + 1) * DH] for i in range(NCB)]
    nrel = -rel_embed
    rels = [nrel[:, i * DH:(i + 1) * DH] for i in range(NCB)]

    sc = functools.partial(
        pl.kernel,
        out_type=jax.ShapeDtypeStruct((2, NCB, NPAD, DH), jnp.float32),
        mesh=plsc.VectorSubcoreMesh(core_axis_name="c", subcore_axis_name="s",
                                    num_cores=1),
        scratch_types=[
            pltpu.VMEM((NCH, C), jnp.int32),
            pltpu.VMEM((NCH, C), jnp.int32),
            pltpu.VMEM((NCH, C), jnp.int32),
            pltpu.VMEM((NCH, C), jnp.int32),
            pltpu.VMEM((NCH, C), jnp.float32),
            pltpu.VMEM((C,), jnp.float32),
            pltpu.VMEM((C,), jnp.float32),
            pltpu.VMEM((C, DH), jnp.float32),
            pltpu.VMEM((C,), jnp.float32),
            pltpu.VMEM((640,), jnp.float32),
            pltpu.VMEM_SHARED((NPAD,), jnp.float32),
            pltpu.VMEM_SHARED((NACC, DH), jnp.float32),
            pltpu.SemaphoreType.DMA,
        ],
    )(_sc_body)
    agg = sc(src_a, dst_a, et_a, *xs, *rels)

    agg_in = jnp.concatenate([agg[0, i, :N] for i in range(NCB)], axis=-1)
    agg_out = jnp.concatenate([agg[1, i, :N] for i in range(NCB)], axis=-1)

    B = 1000
    out = pl.pallas_call(
        _tc_node_body,
        grid=(N // B,),
        in_specs=[
            pl.BlockSpec((B, D), lambda i: (i, 0)),
            pl.BlockSpec((B, D), lambda i: (i, 0)),
            pl.BlockSpec((B, D), lambda i: (i, 0)),
            pl.BlockSpec((D, D), lambda i: (0, 0)),
            pl.BlockSpec((D, D), lambda i: (0, 0)),
            pl.BlockSpec((D, D), lambda i: (0, 0)),
            pl.BlockSpec((1, D), lambda i: (0, 0)),
            pl.BlockSpec((1, D), lambda i: (0, 0)),
        ],
        out_specs=pl.BlockSpec((B, D), lambda i: (i, 0)),
        out_shape=jax.ShapeDtypeStruct((N, D), jnp.float32),
    )(agg_in, agg_out, x, W_in, W_out, W_loop, loop_rel,
      bias.reshape(1, D))

    r_out = pl.pallas_call(
        _tc_rel_body,
        out_shape=jax.ShapeDtypeStruct((RROWS, D), jnp.float32),
    )(rel_embed, W_rel)

    return out, r_out


# speedup vs baseline: 2.0043x; 1.4129x over previous
import functools

import jax
import jax.numpy as jnp
from jax import lax
from jax.experimental import pallas as pl
from jax.experimental.pallas import tpu as pltpu
from jax.experimental.pallas import tpu_sc as plsc

N = 10000
NPAD = 10240
NR = 5120
NACC = 5248
D = 256
DH = 128
NCB = 2
EH = 80000
EPT = 5120
NCH = 40
C = 128
RROWS = 400


def _rsqrt_newton(v):
    i = lax.bitcast_convert_type(v, jnp.int32)
    i = jnp.int32(0x5F3759DF) - (i >> 1)
    y = lax.bitcast_convert_type(i, jnp.float32)
    for _ in range(3):
        y = y * (1.5 - 0.5 * v * y * y)
    return y


def _sc_body(src_hbm, dst_hbm, et_hbm, x0_hbm, x1_hbm, nrel0_hbm, nrel1_hbm,
             agg_hbm,
             src_i, dst_i, et_i, dst2_i, norm_i, nsv, ndv, xrows,
             ones_v, zdeg, deg_sp, acc_sp, sem):
    s = lax.axis_index("s")

    zeros16 = jnp.zeros((16,), jnp.float32)

    def _ov(i, _):
        ones_v[pl.ds(i * 16, 16)] = zeros16 + 1.0
        return 0
    lax.fori_loop(0, C // 16, _ov, 0)

    h = lax.axis_index("c")
    pltpu.sync_copy(src_hbm.at[h, s], src_i)
    pltpu.sync_copy(dst_hbm.at[h, s], dst_i)
    pltpu.sync_copy(et_hbm.at[h, s], et_i)

    def _zv(i, _):
        zdeg[pl.ds(i * 16, 16)] = zeros16
        return 0
    lax.fori_loop(0, 640 // 16, _zv, 0)

    pltpu.sync_copy(zdeg, deg_sp.at[pl.ds(s * 640, 640)])
    plsc.subcore_barrier()

    def _dega(ch, _):
        pltpu.sync_copy(ones_v, deg_sp.at[dst_i.at[ch]], add=True)
        return 0
    lax.fori_loop(0, NCH, _dega, 0)
    plsc.subcore_barrier()

    pltpu.sync_copy(deg_sp.at[pl.ds(s * 640, 640)], zdeg)
    plsc.subcore_barrier()

    def _newt(i, _):
        v = zdeg[pl.ds(i * 16, 16)]
        live = v >= 0.5
        y = _rsqrt_newton(jnp.maximum(v, 1.0))
        zdeg[pl.ds(i * 16, 16)] = jnp.where(live, y, 0.0)
        return 0
    lax.fori_loop(0, 640 // 16, _newt, 0)
    pltpu.sync_copy(zdeg, deg_sp.at[pl.ds(s * 640, 640)])
    plsc.subcore_barrier()

    def _norm(ch, _):
        pltpu.async_copy(deg_sp.at[src_i.at[ch]], nsv, sem).wait()
        pltpu.async_copy(deg_sp.at[dst_i.at[ch]], ndv, sem).wait()
        for j in range(8):
            norm_i[ch, pl.ds(j * 16, 16)] = (
                nsv[pl.ds(j * 16, 16)] * ndv[pl.ds(j * 16, 16)])
        return 0
    lax.fori_loop(0, NCH, _norm, 0)

    for cb in range(NCB):
        xh = (x0_hbm, x1_hbm)[cb]
        nrelh = (nrel0_hbm, nrel1_hbm)[cb]

        def _pass(p, _0):
            lo = p * NR

            def _remap(i, _):
                ch = i // 8
                j = i % 8
                dv = dst_i[ch, pl.ds(j * 16, 16)] - lo
                ok = (dv >= 0) & (dv < NR)
                dst2_i[ch, pl.ds(j * 16, 16)] = jnp.where(ok, dv, NR)
                return 0
            lax.fori_loop(0, NCH * 8, _remap, 0)

            def _zx(i, _):
                xrows[i // 8, pl.ds((i % 8) * 16, 16)] = zeros16
                return 0
            lax.fori_loop(0, C * 8, _zx, 0)
            pltpu.sync_copy(xrows, acc_sp.at[pl.ds(s * C, C)])
            pltpu.sync_copy(xrows, acc_sp.at[pl.ds((s + 16) * C, C)])

            @pl.when(s < 9)
            def _():
                pltpu.sync_copy(xrows, acc_sp.at[pl.ds((s + 32) * C, C)])
            plsc.subcore_barrier()

            def _chunk(ch, _):
                pltpu.async_copy(xh.at[src_i.at[ch]], xrows, sem).wait()
                pltpu.async_copy(nrelh.at[et_i.at[ch]], xrows, sem,
                                 add=True).wait()

                def _grp(g, _2):
                    nv = norm_i[ch, pl.ds(g * 16, 16)]
                    for eo in range(16):
                        nrm = nv[eo]
                        row = g * 16 + eo
                        for j in range(8):
                            xv = xrows[row, pl.ds(j * 16, 16)]
                            xrows[row, pl.ds(j * 16, 16)] = xv * nrm
                    return 0
                lax.fori_loop(0, 8, _grp, 0)

                pltpu.sync_copy(xrows, acc_sp.at[dst2_i.at[ch]], add=True)
                return 0
            lax.fori_loop(0, NCH, _chunk, 0)
            plsc.subcore_barrier()

            pltpu.sync_copy(
                acc_sp.at[pl.ds(s * 320, 320)],
                agg_hbm.at[h, cb, pl.ds(lo + s * 320, 320)])
            plsc.subcore_barrier()
            return 0
        lax.fori_loop(0, 2, _pass, 0)


def _tc_node_body(ain_ref, aout_ref, x_ref, win_ref, wout_ref, wloop_ref,
                  lr_ref, b_ref, out_ref):
    acc = jnp.dot(ain_ref[...], win_ref[...],
                  preferred_element_type=jnp.float32)
    acc += jnp.dot(aout_ref[...], wout_ref[...],
                   preferred_element_type=jnp.float32)
    acc += jnp.dot(x_ref[...] - lr_ref[...], wloop_ref[...],
                   preferred_element_type=jnp.float32)
    out_ref[...] = jnp.tanh(acc * (1.0 / 3.0) + b_ref[...])


def _tc_rel_body(rel_ref, w_ref, out_ref):
    out_ref[...] = jnp.dot(rel_ref[...], w_ref[...],
                           preferred_element_type=jnp.float32)


def kernel(x, edge_index, edge_type, rel_embed, loop_rel, W_in, W_out,
           W_loop, W_rel, bias):
    half = EH

    def prep(idx, pad):
        a = idx.reshape(2, 16, EH // 16)
        p = jnp.full((2, 16, EPT - EH // 16), pad, jnp.int32)
        return jnp.concatenate([a, p], axis=-1).reshape(2, 16, NCH, C)

    src_a = prep(edge_index[0].reshape(2, half), 0)
    dst_a = prep(edge_index[1].reshape(2, half), N)
    et_a = prep(edge_type.reshape(2, half), 0)

    xs = [x[:, i * DH:(i + 1) * DH] for i in range(NCB)]
    nrel = -rel_embed
    rels = [nrel[:, i * DH:(i + 1) * DH] for i in range(NCB)]

    sc = functools.partial(
        pl.kernel,
        out_type=jax.ShapeDtypeStruct((2, NCB, NPAD, DH), jnp.float32),
        mesh=plsc.VectorSubcoreMesh(core_axis_name="c", subcore_axis_name="s"),
        scratch_types=[
            pltpu.VMEM((NCH, C), jnp.int32),
            pltpu.VMEM((NCH, C), jnp.int32),
            pltpu.VMEM((NCH, C), jnp.int32),
            pltpu.VMEM((NCH, C), jnp.int32),
            pltpu.VMEM((NCH, C), jnp.float32),
            pltpu.VMEM((C,), jnp.float32),
            pltpu.VMEM((C,), jnp.float32),
            pltpu.VMEM((C, DH), jnp.float32),
            pltpu.VMEM((C,), jnp.float32),
            pltpu.VMEM((640,), jnp.float32),
            pltpu.VMEM_SHARED((NPAD,), jnp.float32),
            pltpu.VMEM_SHARED((NACC, DH), jnp.float32),
            pltpu.SemaphoreType.DMA,
        ],
    )(_sc_body)
    agg = sc(src_a, dst_a, et_a, *xs, *rels)

    agg_in = jnp.concatenate([agg[0, i, :N] for i in range(NCB)], axis=-1)
    agg_out = jnp.concatenate([agg[1, i, :N] for i in range(NCB)], axis=-1)

    B = 1000
    out = pl.pallas_call(
        _tc_node_body,
        grid=(N // B,),
        in_specs=[
            pl.BlockSpec((B, D), lambda i: (i, 0)),
            pl.BlockSpec((B, D), lambda i: (i, 0)),
            pl.BlockSpec((B, D), lambda i: (i, 0)),
            pl.BlockSpec((D, D), lambda i: (0, 0)),
            pl.BlockSpec((D, D), lambda i: (0, 0)),
            pl.BlockSpec((D, D), lambda i: (0, 0)),
            pl.BlockSpec((1, D), lambda i: (0, 0)),
            pl.BlockSpec((1, D), lambda i: (0, 0)),
        ],
        out_specs=pl.BlockSpec((B, D), lambda i: (i, 0)),
        out_shape=jax.ShapeDtypeStruct((N, D), jnp.float32),
    )(agg_in, agg_out, x, W_in, W_out, W_loop, loop_rel,
      bias.reshape(1, D))

    r_out = pl.pallas_call(
        _tc_rel_body,
        out_shape=jax.ShapeDtypeStruct((RROWS, D), jnp.float32),
    )(rel_embed, W_rel)

    return out, r_out
